# baseline (speedup 1.0000x reference)
import functools

import jax
import jax.numpy as jnp
from jax import lax
from jax.experimental import pallas as pl
from jax.experimental.pallas import tpu as pltpu
from jax.experimental.pallas import tpu_sc as plsc

H = 128
L = 128
B = 4096
V0 = 500
V1 = 2000
N_TOPO = 400000
N_CLS = 10000
N_ASSM = 400000

NC, NS = 2, 16
NW = NC * NS
CH = 128
G_CH = 3
ROWS_G = CH * G_CH
N_CHUNKS = N_TOPO // CH
N_FULL_GROUPS = N_CHUNKS // G_CH
TAIL_CHUNKS = N_CHUNKS - N_FULL_GROUPS * G_CH
CLS_W = 80
N_CLS_GROUPS = N_CLS // CLS_W

_MESH = dict(core_axis_name="c", subcore_axis_name="s",
             num_cores=NC, num_subcores=NS)

_NT = (((1,), (1,)), ((), ()))


def _pipe(idx_h, table_h, out_h, bufs, wid, n_groups, chunks):
  rows_g = sum(chunks)
  offs = [sum(chunks[:j]) for j in range(len(chunks))]
  iters = -(-n_groups // NW)

  def fire(t, slot):
    idx_v, rows_v, sem = bufs[slot]
    g = wid + t * NW

    @pl.when(g < n_groups)
    def _():
      pltpu.sync_copy(idx_h.at[pl.ds(g * rows_g, rows_g)], idx_v)
      for o, c in zip(offs, chunks):
        pltpu.async_copy(table_h.at[idx_v.at[pl.ds(o, c)]],
                         rows_v.at[pl.ds(o, c)], sem)

  def complete(t, slot):
    idx_v, rows_v, sem = bufs[slot]
    g = wid + t * NW

    @pl.when(g < n_groups)
    def _():
      for o, c in zip(offs, chunks):
        pltpu.make_async_copy(table_h.at[idx_v.at[pl.ds(o, c)]],
                              rows_v.at[pl.ds(o, c)], sem).wait()
      pltpu.sync_copy(rows_v, out_h.at[pl.ds(g * rows_g, rows_g)])

  fire(0, 0)

  def body(t2, carry):
    fire(2 * t2 + 1, 1)
    complete(2 * t2, 0)
    fire(2 * t2 + 2, 0)
    complete(2 * t2 + 1, 1)
    return carry

  lax.fori_loop(0, (iters - 1) // 2, body, 0)
  if iters % 2 == 1:
    complete(iters - 1, (iters - 1) % 2)
  else:
    fire(iters - 1, (iters - 1) % 2)
    complete(iters - 2, (iters - 2) % 2)
    complete(iters - 1, (iters - 1) % 2)


def _sc_gather_big(table, idx, name):
  n = idx.shape[0]
  n_groups = n // ROWS_G
  tail_chunks = (n - n_groups * ROWS_G) // CH
  assert n_groups * ROWS_G + tail_chunks * CH == n

  @functools.partial(
      pl.kernel,
      out_type=jax.ShapeDtypeStruct((n, H), jnp.float32),
      mesh=plsc.VectorSubcoreMesh(**_MESH),
      scratch_types=[
          pltpu.VMEM((ROWS_G,), jnp.int32),
          pltpu.VMEM((ROWS_G, H), jnp.float32),
          pltpu.SemaphoreType.DMA,
          pltpu.VMEM((ROWS_G,), jnp.int32),
          pltpu.VMEM((ROWS_G, H), jnp.float32),
          pltpu.SemaphoreType.DMA,
      ],
      name=name,
  )
  def k(table_h, idx_h, out_h, idx_v0, rows_v0, sem0, idx_v1, rows_v1, sem1):
    wid = lax.axis_index("s") * NC + lax.axis_index("c")
    bufs = ((idx_v0, rows_v0, sem0), (idx_v1, rows_v1, sem1))
    _pipe(idx_h, table_h, out_h, bufs, wid, n_groups, [CH] * G_CH)

    if tail_chunks:
      @pl.when(wid < tail_chunks)
      def _():
        idx_v, rows_v, sem = bufs[0]
        c = n_groups * G_CH + wid
        pltpu.sync_copy(idx_h.at[pl.ds(c * CH, CH)], idx_v.at[pl.ds(0, CH)])
        pltpu.async_copy(table_h.at[idx_v.at[pl.ds(0, CH)]],
                         rows_v.at[pl.ds(0, CH)], sem).wait()
        pltpu.sync_copy(rows_v.at[pl.ds(0, CH)], out_h.at[pl.ds(c * CH, CH)])

  return k(table, idx)


def _sc_gather_cls(table, cls_idx):
  @functools.partial(
      pl.kernel,
      out_type=jax.ShapeDtypeStruct((N_CLS, H), jnp.float32),
      mesh=plsc.VectorSubcoreMesh(**_MESH),
      scratch_types=[
          pltpu.VMEM((CLS_W,), jnp.int32),
          pltpu.VMEM((CLS_W, H), jnp.float32),
          pltpu.SemaphoreType.DMA,
          pltpu.VMEM((CLS_W,), jnp.int32),
          pltpu.VMEM((CLS_W, H), jnp.float32),
          pltpu.SemaphoreType.DMA,
      ],
      name="sc_gather_cls",
  )
  def k(tab_h, idx_h, out_h, idx_v0, rows_v0, sem0, idx_v1, rows_v1, sem1):
    wid = lax.axis_index("s") * NC + lax.axis_index("c")
    bufs = ((idx_v0, rows_v0, sem0), (idx_v1, rows_v1, sem1))
    _pipe(idx_h, tab_h, out_h, bufs, wid, N_CLS_GROUPS, [CLS_W])

  return k(table, cls_idx)


def _topo_body(tv_ref, g_ref, w1a_ref, w1b_ref, b1_ref, w2t_ref, b2_ref,
               o_ref):
  h = jnp.dot(tv_ref[...].astype(jnp.bfloat16), w1a_ref[...],
              preferred_element_type=jnp.float32)
  h += jnp.dot(g_ref[...].astype(jnp.bfloat16), w1b_ref[...],
               preferred_element_type=jnp.float32)
  h = jnp.maximum(h + b1_ref[...], 0.0)
  s = lax.dot_general(w2t_ref[...].astype(jnp.bfloat16),
                      h.astype(jnp.bfloat16), _NT,
                      preferred_element_type=jnp.float32) + b2_ref[...]
  o_ref[...] = s.reshape(o_ref.shape)


def _assm_body(av_ref, g_ref, w1_ref, b1_ref, ones_ref, b2_ref, o_ref):
  del b2_ref
  p = jnp.dot(av_ref[...].astype(jnp.bfloat16), w1_ref[...],
              preferred_element_type=jnp.float32)
  p = (p + b1_ref[...]) * g_ref[...]
  s = lax.dot_general(ones_ref[...], p.astype(jnp.bfloat16), _NT,
                      preferred_element_type=jnp.float32)
  o_ref[...] = s.reshape(o_ref.shape)


def _cls_body(cv_ref, g_ref, wc1a_ref, wc1b_ref, bc1_ref, wc2t_ref, bc2t_ref,
              wi1a_ref, wi1b_ref, bi1_ref, wi2t_ref, bi2t_ref,
              oct_ref, oit_ref):
  cv = cv_ref[...].astype(jnp.bfloat16)
  g = g_ref[...].astype(jnp.bfloat16)
  hc = jnp.dot(cv, wc1a_ref[...], preferred_element_type=jnp.float32)
  hc += jnp.dot(g, wc1b_ref[...], preferred_element_type=jnp.float32)
  hc = jnp.maximum(hc + bc1_ref[...], 0.0)
  oct_ref[...] = lax.dot_general(
      wc2t_ref[...], hc.astype(jnp.bfloat16), _NT,
      preferred_element_type=jnp.float32) + bc2t_ref[...]
  hi = jnp.dot(cv, wi1a_ref[...], preferred_element_type=jnp.float32)
  hi += jnp.dot(g, wi1b_ref[...], preferred_element_type=jnp.float32)
  hi = jnp.maximum(hi + bi1_ref[...], 0.0)
  oit_ref[...] = lax.dot_general(
      wi2t_ref[...], hi.astype(jnp.bfloat16), _NT,
      preferred_element_type=jnp.float32) + bi2t_ref[...]


BLK_BIG = 20480
BLK_CLS = 1024


def _score_big(vecs, gathered, weights, body, blk_off=0):
  n = gathered.shape[0]
  return pl.pallas_call(
      body,
      grid=(-(-n // BLK_BIG),),
      in_specs=[pl.BlockSpec((BLK_BIG, H), lambda i: (i + blk_off, 0)),
                pl.BlockSpec((BLK_BIG, H), lambda i: (i, 0))] +
               [pl.BlockSpec(w.shape, lambda i: (0,) * w.ndim)
                for w in weights],
      out_specs=pl.BlockSpec((BLK_BIG,), lambda i: (i,)),
      out_shape=jax.ShapeDtypeStruct((n,), jnp.float32),
  )(vecs, gathered, *weights)


def kernel(src_tree_vecs, src_graph_vecs, topo_vecs, cls_vecs, assm_vecs,
           topo_idx, cls_idx, assm_idx,
           W_topo1, b_topo1, W_topo2, b_topo2,
           W_cls1, b_cls1, W_cls2, b_cls2,
           W_icls1, b_icls1, W_icls2, b_icls2,
           W_assm, b_assm):
  g_cls = _sc_gather_cls(src_tree_vecs, cls_idx)
  dep = (g_cls[0, 0] * 0.0).astype(jnp.int32)
  tidx = topo_idx + dep
  aidx = assm_idx + dep
  HALF = 204800
  g_topo1 = _sc_gather_big(src_tree_vecs, tidx[:HALF], "sc_gather_topo1")
  g_assm1 = _sc_gather_big(src_graph_vecs, aidx[:HALF], "sc_gather_assm1")
  g_topo2 = _sc_gather_big(src_tree_vecs, tidx[HALF:], "sc_gather_topo2")
  g_assm2 = _sc_gather_big(src_graph_vecs, aidx[HALF:], "sc_gather_assm2")

  bf = jnp.bfloat16
  nblk_cls = -(-N_CLS // BLK_CLS)
  cls_t, icls_t = pl.pallas_call(
      _cls_body,
      grid=(nblk_cls,),
      in_specs=[
          pl.BlockSpec((BLK_CLS, H), lambda i: (i, 0)),
          pl.BlockSpec((BLK_CLS, H), lambda i: (i, 0)),
          pl.BlockSpec((H, H), lambda i: (0, 0)),
          pl.BlockSpec((H, H), lambda i: (0, 0)),
          pl.BlockSpec((1, H), lambda i: (0, 0)),
          pl.BlockSpec((V0, H), lambda i: (0, 0)),
          pl.BlockSpec((V0, 1), lambda i: (0, 0)),
          pl.BlockSpec((H, H), lambda i: (0, 0)),
          pl.BlockSpec((H, H), lambda i: (0, 0)),
          pl.BlockSpec((1, H), lambda i: (0, 0)),
          pl.BlockSpec((V1, H), lambda i: (0, 0)),
          pl.BlockSpec((V1, 1), lambda i: (0, 0)),
      ],
      out_specs=[
          pl.BlockSpec((V0, BLK_CLS), lambda i: (0, i)),
          pl.BlockSpec((V1, BLK_CLS), lambda i: (0, i)),
      ],
      out_shape=[
          jax.ShapeDtypeStruct((V0, N_CLS), jnp.float32),
          jax.ShapeDtypeStruct((V1, N_CLS), jnp.float32),
      ],
  )(cls_vecs, g_cls, W_cls1[:H].astype(bf), W_cls1[H:].astype(bf),
    b_cls1.reshape(1, H), W_cls2.T.astype(bf), b_cls2.reshape(V0, 1),
    W_icls1[:H].astype(bf), W_icls1[H:].astype(bf), b_icls1.reshape(1, H),
    W_icls2.T.astype(bf), b_icls2.reshape(V1, 1))

  topo_w = (W_topo1[:H].astype(bf), W_topo1[H:].astype(bf),
            b_topo1.reshape(1, H), W_topo2.reshape(1, H),
            b_topo2.reshape(1, 1))
  assm_w = (W_assm.astype(bf), b_assm.reshape(1, L), jnp.ones((1, L), bf),
            jnp.zeros((1, 1), jnp.float32))

  topo1 = _score_big(topo_vecs, g_topo1, topo_w, _topo_body)
  assm1 = _score_big(assm_vecs, g_assm1, assm_w, _assm_body)
  topo2 = _score_big(topo_vecs, g_topo2, topo_w, _topo_body, blk_off=204800 // BLK_BIG)
  assm2 = _score_big(assm_vecs, g_assm2, assm_w, _assm_body, blk_off=204800 // BLK_BIG)
  topo_scores = jnp.concatenate([topo1, topo2])
  assm_scores = jnp.concatenate([assm1, assm2])

  return (topo_scores, cls_t.T, icls_t.T, assm_scores)


# speedup vs baseline: 1.0055x; 1.0055x over previous
import functools

import jax
import jax.numpy as jnp
from jax import lax
from jax.experimental import pallas as pl
from jax.experimental.pallas import tpu as pltpu
from jax.experimental.pallas import tpu_sc as plsc

H = 128
L = 128
B = 4096
V0 = 500
V1 = 2000
N_TOPO = 400000
N_CLS = 10000
N_ASSM = 400000

NC, NS = 2, 16
NW = NC * NS
CH = 128
G_CH = 3
ROWS_G = CH * G_CH
N_CHUNKS = N_TOPO // CH
N_FULL_GROUPS = N_CHUNKS // G_CH
TAIL_CHUNKS = N_CHUNKS - N_FULL_GROUPS * G_CH
CLS_W = 80
N_CLS_GROUPS = N_CLS // CLS_W

_MESH = dict(core_axis_name="c", subcore_axis_name="s",
             num_cores=NC, num_subcores=NS)

_NT = (((1,), (1,)), ((), ()))


def _pipe(idx_h, table_h, out_h, bufs, wid, n_groups, chunks):
  rows_g = sum(chunks)
  offs = [sum(chunks[:j]) for j in range(len(chunks))]
  iters = -(-n_groups // NW)

  def fire(t, slot):
    idx_v, rows_v, sem = bufs[slot]
    g = wid + t * NW

    @pl.when(g < n_groups)
    def _():
      pltpu.sync_copy(idx_h.at[pl.ds(g * rows_g, rows_g)], idx_v)
      for o, c in zip(offs, chunks):
        pltpu.async_copy(table_h.at[idx_v.at[pl.ds(o, c)]],
                         rows_v.at[pl.ds(o, c)], sem)

  def complete(t, slot):
    idx_v, rows_v, sem = bufs[slot]
    g = wid + t * NW

    @pl.when(g < n_groups)
    def _():
      for o, c in zip(offs, chunks):
        pltpu.make_async_copy(table_h.at[idx_v.at[pl.ds(o, c)]],
                              rows_v.at[pl.ds(o, c)], sem).wait()
      pltpu.sync_copy(rows_v, out_h.at[pl.ds(g * rows_g, rows_g)])

  fire(0, 0)

  def body(t2, carry):
    fire(2 * t2 + 1, 1)
    complete(2 * t2, 0)
    fire(2 * t2 + 2, 0)
    complete(2 * t2 + 1, 1)
    return carry

  lax.fori_loop(0, (iters - 1) // 2, body, 0)
  if iters % 2 == 1:
    complete(iters - 1, (iters - 1) % 2)
  else:
    fire(iters - 1, (iters - 1) % 2)
    complete(iters - 2, (iters - 2) % 2)
    complete(iters - 1, (iters - 1) % 2)


def _sc_gather_big(table, idx, name):
  n = idx.shape[0]
  n_groups = n // ROWS_G
  tail_chunks = (n - n_groups * ROWS_G) // CH
  assert n_groups * ROWS_G + tail_chunks * CH == n

  @functools.partial(
      pl.kernel,
      out_type=jax.ShapeDtypeStruct((n, H), jnp.float32),
      mesh=plsc.VectorSubcoreMesh(**_MESH),
      scratch_types=[
          pltpu.VMEM((ROWS_G,), jnp.int32),
          pltpu.VMEM((ROWS_G, H), jnp.float32),
          pltpu.SemaphoreType.DMA,
          pltpu.VMEM((ROWS_G,), jnp.int32),
          pltpu.VMEM((ROWS_G, H), jnp.float32),
          pltpu.SemaphoreType.DMA,
      ],
      name=name,
  )
  def k(table_h, idx_h, out_h, idx_v0, rows_v0, sem0, idx_v1, rows_v1, sem1):
    wid = lax.axis_index("s") * NC + lax.axis_index("c")
    bufs = ((idx_v0, rows_v0, sem0), (idx_v1, rows_v1, sem1))
    _pipe(idx_h, table_h, out_h, bufs, wid, n_groups, [CH] * G_CH)

    if tail_chunks:
      @pl.when(wid < tail_chunks)
      def _():
        idx_v, rows_v, sem = bufs[0]
        c = n_groups * G_CH + wid
        pltpu.sync_copy(idx_h.at[pl.ds(c * CH, CH)], idx_v.at[pl.ds(0, CH)])
        pltpu.async_copy(table_h.at[idx_v.at[pl.ds(0, CH)]],
                         rows_v.at[pl.ds(0, CH)], sem).wait()
        pltpu.sync_copy(rows_v.at[pl.ds(0, CH)], out_h.at[pl.ds(c * CH, CH)])

  return k(table, idx)


def _sc_gather_cls(table, cls_idx):
  @functools.partial(
      pl.kernel,
      out_type=jax.ShapeDtypeStruct((N_CLS, H), jnp.float32),
      mesh=plsc.VectorSubcoreMesh(**_MESH),
      scratch_types=[
          pltpu.VMEM((CLS_W,), jnp.int32),
          pltpu.VMEM((CLS_W, H), jnp.float32),
          pltpu.SemaphoreType.DMA,
          pltpu.VMEM((CLS_W,), jnp.int32),
          pltpu.VMEM((CLS_W, H), jnp.float32),
          pltpu.SemaphoreType.DMA,
      ],
      name="sc_gather_cls",
  )
  def k(tab_h, idx_h, out_h, idx_v0, rows_v0, sem0, idx_v1, rows_v1, sem1):
    wid = lax.axis_index("s") * NC + lax.axis_index("c")
    bufs = ((idx_v0, rows_v0, sem0), (idx_v1, rows_v1, sem1))
    _pipe(idx_h, tab_h, out_h, bufs, wid, N_CLS_GROUPS, [CLS_W])

  return k(table, cls_idx)


def _topo_body(tv_ref, g_ref, w1a_ref, w1b_ref, b1_ref, w2t_ref, b2_ref,
               o_ref):
  h = jnp.dot(tv_ref[...].astype(jnp.bfloat16), w1a_ref[...],
              preferred_element_type=jnp.float32)
  h += jnp.dot(g_ref[...].astype(jnp.bfloat16), w1b_ref[...],
               preferred_element_type=jnp.float32)
  h = jnp.maximum(h + b1_ref[...], 0.0)
  s = lax.dot_general(w2t_ref[...].astype(jnp.bfloat16),
                      h.astype(jnp.bfloat16), _NT,
                      preferred_element_type=jnp.float32) + b2_ref[...]
  o_ref[...] = s.reshape(o_ref.shape)


def _assm_body(av_ref, g_ref, w1_ref, b1_ref, ones_ref, b2_ref, o_ref):
  del b2_ref
  p = jnp.dot(av_ref[...].astype(jnp.bfloat16), w1_ref[...],
              preferred_element_type=jnp.float32)
  p = (p + b1_ref[...]) * g_ref[...]
  s = lax.dot_general(ones_ref[...], p.astype(jnp.bfloat16), _NT,
                      preferred_element_type=jnp.float32)
  o_ref[...] = s.reshape(o_ref.shape)


def _cls_body(cv_ref, g_ref, wc1a_ref, wc1b_ref, bc1_ref, wc2t_ref, bc2t_ref,
              wi1a_ref, wi1b_ref, bi1_ref, wi2t_ref, bi2t_ref,
              oct_ref, oit_ref):
  cv = cv_ref[...].astype(jnp.bfloat16)
  g = g_ref[...].astype(jnp.bfloat16)
  hc = jnp.dot(cv, wc1a_ref[...], preferred_element_type=jnp.float32)
  hc += jnp.dot(g, wc1b_ref[...], preferred_element_type=jnp.float32)
  hc = jnp.maximum(hc + bc1_ref[...], 0.0)
  oct_ref[...] = lax.dot_general(
      wc2t_ref[...], hc.astype(jnp.bfloat16), _NT,
      preferred_element_type=jnp.float32) + bc2t_ref[...]
  hi = jnp.dot(cv, wi1a_ref[...], preferred_element_type=jnp.float32)
  hi += jnp.dot(g, wi1b_ref[...], preferred_element_type=jnp.float32)
  hi = jnp.maximum(hi + bi1_ref[...], 0.0)
  oit_ref[...] = lax.dot_general(
      wi2t_ref[...], hi.astype(jnp.bfloat16), _NT,
      preferred_element_type=jnp.float32) + bi2t_ref[...]


BLK_BIG = 10240
BLK_CLS = 1024


def _score_big(vecs, gathered, weights, body, blk_off=0):
  n = gathered.shape[0]
  return pl.pallas_call(
      body,
      grid=(-(-n // BLK_BIG),),
      in_specs=[pl.BlockSpec((BLK_BIG, H), lambda i: (i + blk_off, 0)),
                pl.BlockSpec((BLK_BIG, H), lambda i: (i, 0))] +
               [pl.BlockSpec(w.shape, lambda i: (0,) * w.ndim)
                for w in weights],
      out_specs=pl.BlockSpec((BLK_BIG,), lambda i: (i,)),
      out_shape=jax.ShapeDtypeStruct((n,), jnp.float32),
  )(vecs, gathered, *weights)


def kernel(src_tree_vecs, src_graph_vecs, topo_vecs, cls_vecs, assm_vecs,
           topo_idx, cls_idx, assm_idx,
           W_topo1, b_topo1, W_topo2, b_topo2,
           W_cls1, b_cls1, W_cls2, b_cls2,
           W_icls1, b_icls1, W_icls2, b_icls2,
           W_assm, b_assm):
  g_cls = _sc_gather_cls(src_tree_vecs, cls_idx)
  dep = (g_cls[0, 0] * 0.0).astype(jnp.int32)
  tidx = topo_idx + dep
  aidx = assm_idx + dep
  HALF = 204800
  g_topo1 = _sc_gather_big(src_tree_vecs, tidx[:HALF], "sc_gather_topo1")
  g_assm1 = _sc_gather_big(src_graph_vecs, aidx[:HALF], "sc_gather_assm1")
  g_topo2 = _sc_gather_big(src_tree_vecs, tidx[HALF:], "sc_gather_topo2")
  g_assm2 = _sc_gather_big(src_graph_vecs, aidx[HALF:], "sc_gather_assm2")

  bf = jnp.bfloat16
  nblk_cls = -(-N_CLS // BLK_CLS)
  cls_t, icls_t = pl.pallas_call(
      _cls_body,
      grid=(nblk_cls,),
      in_specs=[
          pl.BlockSpec((BLK_CLS, H), lambda i: (i, 0)),
          pl.BlockSpec((BLK_CLS, H), lambda i: (i, 0)),
          pl.BlockSpec((H, H), lambda i: (0, 0)),
          pl.BlockSpec((H, H), lambda i: (0, 0)),
          pl.BlockSpec((1, H), lambda i: (0, 0)),
          pl.BlockSpec((V0, H), lambda i: (0, 0)),
          pl.BlockSpec((V0, 1), lambda i: (0, 0)),
          pl.BlockSpec((H, H), lambda i: (0, 0)),
          pl.BlockSpec((H, H), lambda i: (0, 0)),
          pl.BlockSpec((1, H), lambda i: (0, 0)),
          pl.BlockSpec((V1, H), lambda i: (0, 0)),
          pl.BlockSpec((V1, 1), lambda i: (0, 0)),
      ],
      out_specs=[
          pl.BlockSpec((V0, BLK_CLS), lambda i: (0, i)),
          pl.BlockSpec((V1, BLK_CLS), lambda i: (0, i)),
      ],
      out_shape=[
          jax.ShapeDtypeStruct((V0, N_CLS), jnp.float32),
          jax.ShapeDtypeStruct((V1, N_CLS), jnp.float32),
      ],
  )(cls_vecs, g_cls, W_cls1[:H].astype(bf), W_cls1[H:].astype(bf),
    b_cls1.reshape(1, H), W_cls2.T.astype(bf), b_cls2.reshape(V0, 1),
    W_icls1[:H].astype(bf), W_icls1[H:].astype(bf), b_icls1.reshape(1, H),
    W_icls2.T.astype(bf), b_icls2.reshape(V1, 1))

  topo_w = (W_topo1[:H].astype(bf), W_topo1[H:].astype(bf),
            b_topo1.reshape(1, H), W_topo2.reshape(1, H),
            b_topo2.reshape(1, 1))
  assm_w = (W_assm.astype(bf), b_assm.reshape(1, L), jnp.ones((1, L), bf),
            jnp.zeros((1, 1), jnp.float32))

  topo1 = _score_big(topo_vecs, g_topo1, topo_w, _topo_body)
  assm1 = _score_big(assm_vecs, g_assm1, assm_w, _assm_body)
  topo2 = _score_big(topo_vecs, g_topo2, topo_w, _topo_body, blk_off=204800 // BLK_BIG)
  assm2 = _score_big(assm_vecs, g_assm2, assm_w, _assm_body, blk_off=204800 // BLK_BIG)
  topo_scores = jnp.concatenate([topo1, topo2])
  assm_scores = jnp.concatenate([assm1, assm2])

  return (topo_scores, cls_t.T, icls_t.T, assm_scores)
